# baseline (speedup 1.0000x reference)
import jax
import jax.numpy as jnp
from jax import lax
from jax.experimental import pallas as pl
from jax.experimental.pallas import tpu as pltpu
from jax.experimental.pallas import tpu_sc as plsc

N = 10000
E = 160000
D = 256
DH = 128
DE = 16
DEA = 128
DOUT = 256

NTILES = 16
EPT = E // NTILES
B = 128
EPTP = 10240
NBLK = EPTP // B
BE = 80
NBE = EPT // BE
RPT = 632


LAST = N - (NTILES - 1) * RPT


def _rowcopy(s, src_slot, dst_slot):
    @pl.when(s < NTILES - 1)
    def _():
        pltpu.sync_copy(src_slot(s * RPT, RPT), dst_slot(s * RPT, RPT))

    @pl.when(s == NTILES - 1)
    def _():
        base = (NTILES - 1) * RPT
        pltpu.sync_copy(src_slot(base, LAST), dst_slot(base, LAST))


def _sc_x_body(srcadj, dsts, xt, zx, out_x, acc_x, srca, dsta, rows, sem):
    c = lax.axis_index("c")
    s = lax.axis_index("s")

    for h in range(2):
        p = 2 * c + h
        _rowcopy(s, lambda o, n: zx.at[pl.ds(0, n)],
                 lambda o, n: acc_x.at[pl.ds(o, n)])
        pltpu.sync_copy(srcadj.at[p, s], srca)
        if h == 0:
            pltpu.sync_copy(dsts.at[c, s], dsta)
        plsc.subcore_barrier()

        def blk(i, carry):
            pltpu.async_copy(
                xt.at[srca.at[pl.ds(i * B, B)]], rows, sem).wait()
            pltpu.sync_copy(rows, acc_x.at[dsta.at[i]], add=True)
            return carry

        lax.fori_loop(0, NBLK, blk, 0)
        plsc.subcore_barrier()
        _rowcopy(s, lambda o, n: acc_x.at[pl.ds(o, n)],
                 lambda o, n: out_x.at[pl.ds(pl.multiple_of(p * N + o, 8), n)])
        plsc.subcore_barrier()


def _sc_e_body(dsts_e, efa, ze, out_e, acc_e, dsti, efb):
    c = lax.axis_index("c")
    s = lax.axis_index("s")

    _rowcopy(s, lambda o, n: ze.at[pl.ds(0, n)],
             lambda o, n: acc_e.at[pl.ds(o, n)])
    plsc.subcore_barrier()

    def blk(i, carry):
        ebase = pl.multiple_of((c * NTILES + s) * EPT + i * BE, 8)
        pltpu.sync_copy(dsts_e.at[pl.ds(ebase, BE)], dsti)
        pltpu.sync_copy(efa.at[pl.ds(ebase, BE)], efb)
        pltpu.sync_copy(efb, acc_e.at[dsti], add=True)
        return carry

    lax.fori_loop(0, NBE, blk, 0)
    plsc.subcore_barrier()
    _rowcopy(s, lambda o, n: acc_e.at[pl.ds(o, n)],
             lambda o, n: out_e.at[pl.ds(pl.multiple_of(c * N + o, 8), n)])


def _sc_segment_sums(src0, dst0, ef0a, x0lo, x0hi,
                     src1, dst1, ef1a, x1lo, x1hi):
    xt = jnp.concatenate(
        [x0lo, x0hi, x1lo, x1hi, jnp.zeros((8, DH), jnp.float32)], axis=0)
    PADC = EPTP - EPT

    def padtiles(idx, fill):
        t = idx.reshape(-1, NTILES, EPT)
        return jnp.pad(t, ((0, 0), (0, 0), (0, PADC)), constant_values=fill)

    srcadj = padtiles(
        jnp.concatenate([src0, src0 + N, src1 + 2 * N, src1 + 3 * N]),
        4 * N)
    dstcat = jnp.concatenate([dst0, dst1])
    dsts = padtiles(dstcat, 0).reshape(2, NTILES, NBLK, B)
    dsts_e = dstcat
    efa = jnp.concatenate([ef0a, ef1a], axis=0)

    zx = jnp.zeros((RPT, DH), jnp.float32)
    ze = jnp.zeros((RPT, DEA), jnp.float32)
    mesh = plsc.VectorSubcoreMesh(core_axis_name="c", subcore_axis_name="s")
    fx = pl.kernel(
        _sc_x_body,
        out_type=jax.ShapeDtypeStruct((4 * N, DH), jnp.float32),
        mesh=mesh,
        scratch_types=[
            pltpu.VMEM_SHARED((N, DH), jnp.float32),
            pltpu.VMEM((EPTP,), jnp.int32),
            pltpu.VMEM((NBLK, B), jnp.int32),
            pltpu.VMEM((B, DH), jnp.float32),
            pltpu.SemaphoreType.DMA,
        ],
    )
    fe = pl.kernel(
        _sc_e_body,
        out_type=jax.ShapeDtypeStruct((2 * N, DEA), jnp.float32),
        mesh=mesh,
        scratch_types=[
            pltpu.VMEM_SHARED((N, DEA), jnp.float32),
            pltpu.VMEM((BE,), jnp.int32),
            pltpu.VMEM((BE, DEA), jnp.float32),
        ],
    )
    return fx(srcadj, dsts, xt, zx), fe(dsts_e, efa, ze)


def _combine_body(wn, ws, wu, bn, bs, w1, w2, we, cv):
    wu_n = wu[0:DOUT, :]
    wu_e = wu[DOUT:DOUT + DE, :]
    wu_s = wu[DOUT + DE:, :]
    w1[...] = jnp.dot(wn[...], wu_n, preferred_element_type=jnp.float32)
    w2[...] = jnp.dot(ws[...], wu_s, preferred_element_type=jnp.float32)
    we[...] = wu_e
    cv[...] = (jnp.dot(bn[...], wu_n, preferred_element_type=jnp.float32)
               + jnp.dot(bs[...], wu_s, preferred_element_type=jnp.float32))


def _combine(wn, ws, wu, bn, bs):
    return pl.pallas_call(
        _combine_body,
        out_shape=(
            jax.ShapeDtypeStruct((D, DOUT), jnp.float32),
            jax.ShapeDtypeStruct((D, DOUT), jnp.float32),
            jax.ShapeDtypeStruct((DE, DOUT), jnp.float32),
            jax.ShapeDtypeStruct((1, DOUT), jnp.float32),
        ),
    )(wn, ws, wu, bn.reshape(1, D), bs.reshape(1, D))


RB = 400


def _out_body(sx_lo, sx_hi, sea, xs, w1, w2, we, cv, bu, out):
    se = sea[:, 0:DE]
    deg = sea[:, DE:DE + 1]
    acc = jnp.dot(sx_lo[...], w1[0:DH, :], preferred_element_type=jnp.float32)
    acc += jnp.dot(sx_hi[...], w1[DH:, :], preferred_element_type=jnp.float32)
    acc += jnp.dot(xs[...] * deg, w2[...], preferred_element_type=jnp.float32)
    acc += jnp.dot(se, we[...], preferred_element_type=jnp.float32)
    acc += deg * cv[...]
    acc += bu[...]
    out[...] = acc


def _dense_out(sx_lo, sx_hi, sea, xs, w1, w2, we, cv, bu):
    grid = (N // RB,)
    row = lambda i: (i, 0)
    fix = lambda i: (0, 0)
    return pl.pallas_call(
        _out_body,
        grid=grid,
        in_specs=[
            pl.BlockSpec((RB, DH), row),
            pl.BlockSpec((RB, DH), row),
            pl.BlockSpec((RB, DEA), row),
            pl.BlockSpec((RB, D), row),
            pl.BlockSpec((D, DOUT), fix),
            pl.BlockSpec((D, DOUT), fix),
            pl.BlockSpec((DE, DOUT), fix),
            pl.BlockSpec((1, DOUT), fix),
            pl.BlockSpec((1, DOUT), fix),
        ],
        out_specs=pl.BlockSpec((RB, DOUT), row),
        out_shape=jax.ShapeDtypeStruct((N, DOUT), jnp.float32),
    )(sx_lo, sx_hi, sea, xs, w1, w2, we, cv, bu)


def kernel(x_n0, x_n1, edge_index_0, edge_feature_0,
           edge_index_1, edge_feature_1,
           Wn0, bn0, Ws0, bs0, Wu0, bu0,
           Wn1, bn1, Ws1, bs1, Wu1, bu1):
    src0 = edge_index_0[0].astype(jnp.int32)
    dst0 = edge_index_0[1].astype(jnp.int32)
    src1 = edge_index_1[0].astype(jnp.int32)
    dst1 = edge_index_1[1].astype(jnp.int32)

    x0lo, x0hi = x_n0[:, :DH], x_n0[:, DH:]
    x1lo, x1hi = x_n1[:, :DH], x_n1[:, DH:]

    pad = jnp.concatenate(
        [jnp.ones((E, 1), jnp.float32), jnp.zeros((E, DEA - DE - 1), jnp.float32)],
        axis=1)
    ef0a = jnp.concatenate([edge_feature_0, pad], axis=1)
    ef1a = jnp.concatenate([edge_feature_1, pad], axis=1)

    sx, sea = _sc_segment_sums(src0, dst0, ef0a, x0lo, x0hi,
                               src1, dst1, ef1a, x1lo, x1hi)

    w10, w20, we0, cv0 = _combine(Wn0, Ws0, Wu0, bn0, bs0)
    w11, w21, we1, cv1 = _combine(Wn1, Ws1, Wu1, bn1, bs1)

    emb_n1 = _dense_out(sx[0:N], sx[N:2 * N], sea[0:N], x_n1,
                        w10, w20, we0, cv0, bu0.reshape(1, DOUT))
    emb_n0 = _dense_out(sx[2 * N:3 * N], sx[3 * N:], sea[N:], x_n0,
                        w11, w21, we1, cv1, bu1.reshape(1, DOUT))
    return (emb_n0, emb_n1)


# speedup vs baseline: 1.6356x; 1.6356x over previous
import jax
import jax.numpy as jnp
from jax import lax
from jax.experimental import pallas as pl
from jax.experimental.pallas import tpu as pltpu
from jax.experimental.pallas import tpu_sc as plsc

N = 10000
E = 160000
D = 256
DH = 128
DE = 16
DEA = 128
DOUT = 256

NTILES = 16
EPT = E // NTILES
B = 80
NBLK = EPT // B
RPT = 632


LAST = N - (NTILES - 1) * RPT


def _rowcopy(s, src_slot, dst_slot):
    @pl.when(s < NTILES - 1)
    def _():
        pltpu.sync_copy(src_slot(s * RPT, RPT), dst_slot(s * RPT, RPT))

    @pl.when(s == NTILES - 1)
    def _():
        base = (NTILES - 1) * RPT
        pltpu.sync_copy(src_slot(base, LAST), dst_slot(base, LAST))


def _sc_x_body(srcadj, dsts, xt, zx, out_x, acc_x,
               srci0, srci1, dsti0, dsti1, rows0, rows1, sem0, sem1):
    c = lax.axis_index("c")
    s = lax.axis_index("s")

    for h in range(2):
        p = 2 * c + h
        _rowcopy(s, lambda o, n: zx.at[pl.ds(0, n)],
                 lambda o, n: acc_x.at[pl.ds(o, n)])
        plsc.subcore_barrier()

        slots = ((srci0, dsti0, rows0, sem0), (srci1, dsti1, rows1, sem1))

        def start(i, slot):
            srci, _, rows, sem = slots[slot]
            base = pl.multiple_of(p * E + s * EPT + i * B, 8)
            pltpu.sync_copy(srcadj.at[pl.ds(base, B)], srci)
            pltpu.async_copy(xt.at[srci], rows, sem)

        def finish(i, slot):
            srci, dsti, rows, sem = slots[slot]
            ebase = pl.multiple_of(c * E + s * EPT + i * B, 8)
            pltpu.sync_copy(dsts.at[pl.ds(ebase, B)], dsti)
            pltpu.make_async_copy(xt.at[srci], rows, sem).wait()
            pltpu.sync_copy(rows, acc_x.at[dsti], add=True)

        start(0, 0)

        def blk(j, carry):
            i = 2 * j

            @pl.when(i + 1 < NBLK)
            def _():
                start(i + 1, 1)

            finish(i, 0)

            @pl.when(i + 2 < NBLK)
            def _():
                start(i + 2, 0)

            @pl.when(i + 1 < NBLK)
            def _():
                finish(i + 1, 1)

            return carry

        lax.fori_loop(0, (NBLK + 1) // 2, blk, 0)
        plsc.subcore_barrier()
        _rowcopy(s, lambda o, n: acc_x.at[pl.ds(o, n)],
                 lambda o, n: out_x.at[pl.ds(pl.multiple_of(p * N + o, 8), n)])
        plsc.subcore_barrier()


def _sc_e_body(dsts, efa, ze, out_e, acc_e, dsti, efb):
    c = lax.axis_index("c")
    s = lax.axis_index("s")

    _rowcopy(s, lambda o, n: ze.at[pl.ds(0, n)],
             lambda o, n: acc_e.at[pl.ds(o, n)])
    plsc.subcore_barrier()

    def blk(i, carry):
        ebase = pl.multiple_of(c * E + s * EPT + i * B, 8)
        pltpu.sync_copy(dsts.at[pl.ds(ebase, B)], dsti)
        pltpu.sync_copy(efa.at[pl.ds(ebase, B)], efb)
        pltpu.sync_copy(efb, acc_e.at[dsti], add=True)
        return carry

    lax.fori_loop(0, NBLK, blk, 0)
    plsc.subcore_barrier()
    _rowcopy(s, lambda o, n: acc_e.at[pl.ds(o, n)],
             lambda o, n: out_e.at[pl.ds(pl.multiple_of(c * N + o, 8), n)])


def _sc_segment_sums(src0, dst0, ef0a, x0lo, x0hi,
                     src1, dst1, ef1a, x1lo, x1hi):
    xt = jnp.concatenate([x0lo, x0hi, x1lo, x1hi], axis=0)
    srcadj = jnp.concatenate(
        [src0, src0 + N, src1 + 2 * N, src1 + 3 * N])
    dsts = jnp.concatenate([dst0, dst1])
    efa = jnp.concatenate([ef0a, ef1a], axis=0)

    zx = jnp.zeros((RPT, DH), jnp.float32)
    ze = jnp.zeros((RPT, DEA), jnp.float32)
    mesh = plsc.VectorSubcoreMesh(core_axis_name="c", subcore_axis_name="s")
    fx = pl.kernel(
        _sc_x_body,
        out_type=jax.ShapeDtypeStruct((4 * N, DH), jnp.float32),
        mesh=mesh,
        scratch_types=[
            pltpu.VMEM_SHARED((N, DH), jnp.float32),
            pltpu.VMEM((B,), jnp.int32),
            pltpu.VMEM((B,), jnp.int32),
            pltpu.VMEM((B,), jnp.int32),
            pltpu.VMEM((B,), jnp.int32),
            pltpu.VMEM((B, DH), jnp.float32),
            pltpu.VMEM((B, DH), jnp.float32),
            pltpu.SemaphoreType.DMA,
            pltpu.SemaphoreType.DMA,
        ],
    )
    fe = pl.kernel(
        _sc_e_body,
        out_type=jax.ShapeDtypeStruct((2 * N, DEA), jnp.float32),
        mesh=mesh,
        scratch_types=[
            pltpu.VMEM_SHARED((N, DEA), jnp.float32),
            pltpu.VMEM((B,), jnp.int32),
            pltpu.VMEM((B, DEA), jnp.float32),
        ],
    )
    return fx(srcadj, dsts, xt, zx), fe(dsts, efa, ze)


def _combine_body(wn, ws, wu, bn, bs, w1, w2, we, cv):
    wu_n = wu[0:DOUT, :]
    wu_e = wu[DOUT:DOUT + DE, :]
    wu_s = wu[DOUT + DE:, :]
    w1[...] = jnp.dot(wn[...], wu_n, preferred_element_type=jnp.float32)
    w2[...] = jnp.dot(ws[...], wu_s, preferred_element_type=jnp.float32)
    we[...] = wu_e
    cv[...] = (jnp.dot(bn[...], wu_n, preferred_element_type=jnp.float32)
               + jnp.dot(bs[...], wu_s, preferred_element_type=jnp.float32))


def _combine(wn, ws, wu, bn, bs):
    return pl.pallas_call(
        _combine_body,
        out_shape=(
            jax.ShapeDtypeStruct((D, DOUT), jnp.float32),
            jax.ShapeDtypeStruct((D, DOUT), jnp.float32),
            jax.ShapeDtypeStruct((DE, DOUT), jnp.float32),
            jax.ShapeDtypeStruct((1, DOUT), jnp.float32),
        ),
    )(wn, ws, wu, bn.reshape(1, D), bs.reshape(1, D))


RB = 400


def _out_body(sx_lo, sx_hi, sea, xs, w1, w2, we, cv, bu, out):
    se = sea[:, 0:DE]
    deg = sea[:, DE:DE + 1]
    acc = jnp.dot(sx_lo[...], w1[0:DH, :], preferred_element_type=jnp.float32)
    acc += jnp.dot(sx_hi[...], w1[DH:, :], preferred_element_type=jnp.float32)
    acc += jnp.dot(xs[...] * deg, w2[...], preferred_element_type=jnp.float32)
    acc += jnp.dot(se, we[...], preferred_element_type=jnp.float32)
    acc += deg * cv[...]
    acc += bu[...]
    out[...] = acc


def _dense_out(sx_lo, sx_hi, sea, xs, w1, w2, we, cv, bu):
    grid = (N // RB,)
    row = lambda i: (i, 0)
    fix = lambda i: (0, 0)
    return pl.pallas_call(
        _out_body,
        grid=grid,
        in_specs=[
            pl.BlockSpec((RB, DH), row),
            pl.BlockSpec((RB, DH), row),
            pl.BlockSpec((RB, DEA), row),
            pl.BlockSpec((RB, D), row),
            pl.BlockSpec((D, DOUT), fix),
            pl.BlockSpec((D, DOUT), fix),
            pl.BlockSpec((DE, DOUT), fix),
            pl.BlockSpec((1, DOUT), fix),
            pl.BlockSpec((1, DOUT), fix),
        ],
        out_specs=pl.BlockSpec((RB, DOUT), row),
        out_shape=jax.ShapeDtypeStruct((N, DOUT), jnp.float32),
    )(sx_lo, sx_hi, sea, xs, w1, w2, we, cv, bu)


def kernel(x_n0, x_n1, edge_index_0, edge_feature_0,
           edge_index_1, edge_feature_1,
           Wn0, bn0, Ws0, bs0, Wu0, bu0,
           Wn1, bn1, Ws1, bs1, Wu1, bu1):
    src0 = edge_index_0[0].astype(jnp.int32)
    dst0 = edge_index_0[1].astype(jnp.int32)
    src1 = edge_index_1[0].astype(jnp.int32)
    dst1 = edge_index_1[1].astype(jnp.int32)

    x0lo, x0hi = x_n0[:, :DH], x_n0[:, DH:]
    x1lo, x1hi = x_n1[:, :DH], x_n1[:, DH:]

    pad = jnp.concatenate(
        [jnp.ones((E, 1), jnp.float32), jnp.zeros((E, DEA - DE - 1), jnp.float32)],
        axis=1)
    ef0a = jnp.concatenate([edge_feature_0, pad], axis=1)
    ef1a = jnp.concatenate([edge_feature_1, pad], axis=1)

    sx, sea = _sc_segment_sums(src0, dst0, ef0a, x0lo, x0hi,
                               src1, dst1, ef1a, x1lo, x1hi)

    w10, w20, we0, cv0 = _combine(Wn0, Ws0, Wu0, bn0, bs0)
    w11, w21, we1, cv1 = _combine(Wn1, Ws1, Wu1, bn1, bs1)

    emb_n1 = _dense_out(sx[0:N], sx[N:2 * N], sea[0:N], x_n1,
                        w10, w20, we0, cv0, bu0.reshape(1, DOUT))
    emb_n0 = _dense_out(sx[2 * N:3 * N], sx[3 * N:], sea[N:], x_n0,
                        w11, w21, we1, cv1, bu1.reshape(1, DOUT))
    return (emb_n0, emb_n1)


# speedup vs baseline: 1.8686x; 1.1424x over previous
import jax
import jax.numpy as jnp
from jax import lax
from jax.experimental import pallas as pl
from jax.experimental.pallas import tpu as pltpu
from jax.experimental.pallas import tpu_sc as plsc

N = 10000
E = 160000
D = 256
DH = 128
DE = 16
DEA = 128
DOUT = 256

NTILES = 16
EPT = E // NTILES
B = 80
NBLK = EPT // B
RPT = 632


LAST = N - (NTILES - 1) * RPT


def _rowcopy(s, src_slot, dst_slot):
    @pl.when(s < NTILES - 1)
    def _():
        pltpu.sync_copy(src_slot(s * RPT, RPT), dst_slot(s * RPT, RPT))

    @pl.when(s == NTILES - 1)
    def _():
        base = (NTILES - 1) * RPT
        pltpu.sync_copy(src_slot(base, LAST), dst_slot(base, LAST))


def _sc_x_body(srcadj, dsts, xt, zx, out_x, acc_x,
               srci0, srci1, dsti0, dsti1, rows0, rows1, sem0, sem1):
    c = lax.axis_index("c")
    s = lax.axis_index("s")

    for h in range(2):
        p = 2 * c + h
        _rowcopy(s, lambda o, n: zx.at[pl.ds(0, n)],
                 lambda o, n: acc_x.at[pl.ds(o, n)])
        plsc.subcore_barrier()

        slots = ((srci0, dsti0, rows0, sem0), (srci1, dsti1, rows1, sem1))

        def start(i, slot):
            srci, _, rows, sem = slots[slot]
            base = pl.multiple_of(p * E + s * EPT + i * B, 8)
            pltpu.sync_copy(srcadj.at[pl.ds(base, B)], srci)
            pltpu.async_copy(xt.at[srci], rows, sem)

        def finish(i, slot):
            srci, dsti, rows, sem = slots[slot]
            ebase = pl.multiple_of(c * E + s * EPT + i * B, 8)
            pltpu.sync_copy(dsts.at[pl.ds(ebase, B)], dsti)
            pltpu.make_async_copy(xt.at[srci], rows, sem).wait()
            pltpu.sync_copy(rows, acc_x.at[dsti], add=True)

        start(0, 0)

        def blk(j, carry):
            i = 2 * j

            @pl.when(i + 1 < NBLK)
            def _():
                start(i + 1, 1)

            finish(i, 0)

            @pl.when(i + 2 < NBLK)
            def _():
                start(i + 2, 0)

            @pl.when(i + 1 < NBLK)
            def _():
                finish(i + 1, 1)

            return carry

        lax.fori_loop(0, (NBLK + 1) // 2, blk, 0)
        plsc.subcore_barrier()
        _rowcopy(s, lambda o, n: acc_x.at[pl.ds(o, n)],
                 lambda o, n: out_x.at[pl.ds(pl.multiple_of(p * N + o, 8), n)])
        plsc.subcore_barrier()


def _sc_e_body(dsts, efa, ze, out_e, acc_e,
               dsti0, dsti1, efb0, efb1, sem0, sem1):
    c = lax.axis_index("c")
    s = lax.axis_index("s")

    _rowcopy(s, lambda o, n: ze.at[pl.ds(0, n)],
             lambda o, n: acc_e.at[pl.ds(o, n)])
    plsc.subcore_barrier()

    slots = ((dsti0, efb0, sem0), (dsti1, efb1, sem1))

    def start(i, slot):
        dsti, efb, sem = slots[slot]
        ebase = pl.multiple_of(c * E + s * EPT + i * B, 8)
        pltpu.sync_copy(dsts.at[pl.ds(ebase, B)], dsti)
        pltpu.async_copy(efa.at[pl.ds(ebase, B)], efb, sem)

    def finish(i, slot):
        dsti, efb, sem = slots[slot]
        ebase = pl.multiple_of(c * E + s * EPT + i * B, 8)
        pltpu.make_async_copy(efa.at[pl.ds(ebase, B)], efb, sem).wait()
        pltpu.sync_copy(efb, acc_e.at[dsti], add=True)

    start(0, 0)

    def blk(j, carry):
        i = 2 * j

        @pl.when(i + 1 < NBLK)
        def _():
            start(i + 1, 1)

        finish(i, 0)

        @pl.when(i + 2 < NBLK)
        def _():
            start(i + 2, 0)

        @pl.when(i + 1 < NBLK)
        def _():
            finish(i + 1, 1)

        return carry

    lax.fori_loop(0, (NBLK + 1) // 2, blk, 0)
    plsc.subcore_barrier()
    _rowcopy(s, lambda o, n: acc_e.at[pl.ds(o, n)],
             lambda o, n: out_e.at[pl.ds(pl.multiple_of(c * N + o, 8), n)])


def _sc_segment_sums(src0, dst0, ef0a, x0lo, x0hi,
                     src1, dst1, ef1a, x1lo, x1hi):
    xt = jnp.concatenate([x0lo, x0hi, x1lo, x1hi], axis=0)
    srcadj = jnp.concatenate(
        [src0, src0 + N, src1 + 2 * N, src1 + 3 * N])
    dsts = jnp.concatenate([dst0, dst1])
    efa = jnp.concatenate([ef0a, ef1a], axis=0)

    zx = jnp.zeros((RPT, DH), jnp.float32)
    ze = jnp.zeros((RPT, DEA), jnp.float32)
    mesh = plsc.VectorSubcoreMesh(core_axis_name="c", subcore_axis_name="s")
    fx = pl.kernel(
        _sc_x_body,
        out_type=jax.ShapeDtypeStruct((4 * N, DH), jnp.float32),
        mesh=mesh,
        scratch_types=[
            pltpu.VMEM_SHARED((N, DH), jnp.float32),
            pltpu.VMEM((B,), jnp.int32),
            pltpu.VMEM((B,), jnp.int32),
            pltpu.VMEM((B,), jnp.int32),
            pltpu.VMEM((B,), jnp.int32),
            pltpu.VMEM((B, DH), jnp.float32),
            pltpu.VMEM((B, DH), jnp.float32),
            pltpu.SemaphoreType.DMA,
            pltpu.SemaphoreType.DMA,
        ],
    )
    fe = pl.kernel(
        _sc_e_body,
        out_type=jax.ShapeDtypeStruct((2 * N, DEA), jnp.float32),
        mesh=mesh,
        scratch_types=[
            pltpu.VMEM_SHARED((N, DEA), jnp.float32),
            pltpu.VMEM((B,), jnp.int32),
            pltpu.VMEM((B,), jnp.int32),
            pltpu.VMEM((B, DEA), jnp.float32),
            pltpu.VMEM((B, DEA), jnp.float32),
            pltpu.SemaphoreType.DMA,
            pltpu.SemaphoreType.DMA,
        ],
    )
    return fx(srcadj, dsts, xt, zx), fe(dsts, efa, ze)


def _combine_body(wn, ws, wu, bn, bs, w1, w2, we, cv):
    wu_n = wu[0:DOUT, :]
    wu_e = wu[DOUT:DOUT + DE, :]
    wu_s = wu[DOUT + DE:, :]
    w1[...] = jnp.dot(wn[...], wu_n, preferred_element_type=jnp.float32)
    w2[...] = jnp.dot(ws[...], wu_s, preferred_element_type=jnp.float32)
    we[...] = wu_e
    cv[...] = (jnp.dot(bn[...], wu_n, preferred_element_type=jnp.float32)
               + jnp.dot(bs[...], wu_s, preferred_element_type=jnp.float32))


def _combine(wn, ws, wu, bn, bs):
    return pl.pallas_call(
        _combine_body,
        out_shape=(
            jax.ShapeDtypeStruct((D, DOUT), jnp.float32),
            jax.ShapeDtypeStruct((D, DOUT), jnp.float32),
            jax.ShapeDtypeStruct((DE, DOUT), jnp.float32),
            jax.ShapeDtypeStruct((1, DOUT), jnp.float32),
        ),
    )(wn, ws, wu, bn.reshape(1, D), bs.reshape(1, D))


RB = 400


def _out_body(sx_lo, sx_hi, sea, xs, w1, w2, we, cv, bu, out):
    se = sea[:, 0:DE]
    deg = sea[:, DE:DE + 1]
    acc = jnp.dot(sx_lo[...], w1[0:DH, :], preferred_element_type=jnp.float32)
    acc += jnp.dot(sx_hi[...], w1[DH:, :], preferred_element_type=jnp.float32)
    acc += jnp.dot(xs[...] * deg, w2[...], preferred_element_type=jnp.float32)
    acc += jnp.dot(se, we[...], preferred_element_type=jnp.float32)
    acc += deg * cv[...]
    acc += bu[...]
    out[...] = acc


def _dense_out(sx_lo, sx_hi, sea, xs, w1, w2, we, cv, bu):
    grid = (N // RB,)
    row = lambda i: (i, 0)
    fix = lambda i: (0, 0)
    return pl.pallas_call(
        _out_body,
        grid=grid,
        in_specs=[
            pl.BlockSpec((RB, DH), row),
            pl.BlockSpec((RB, DH), row),
            pl.BlockSpec((RB, DEA), row),
            pl.BlockSpec((RB, D), row),
            pl.BlockSpec((D, DOUT), fix),
            pl.BlockSpec((D, DOUT), fix),
            pl.BlockSpec((DE, DOUT), fix),
            pl.BlockSpec((1, DOUT), fix),
            pl.BlockSpec((1, DOUT), fix),
        ],
        out_specs=pl.BlockSpec((RB, DOUT), row),
        out_shape=jax.ShapeDtypeStruct((N, DOUT), jnp.float32),
    )(sx_lo, sx_hi, sea, xs, w1, w2, we, cv, bu)


def kernel(x_n0, x_n1, edge_index_0, edge_feature_0,
           edge_index_1, edge_feature_1,
           Wn0, bn0, Ws0, bs0, Wu0, bu0,
           Wn1, bn1, Ws1, bs1, Wu1, bu1):
    src0 = edge_index_0[0].astype(jnp.int32)
    dst0 = edge_index_0[1].astype(jnp.int32)
    src1 = edge_index_1[0].astype(jnp.int32)
    dst1 = edge_index_1[1].astype(jnp.int32)

    x0lo, x0hi = x_n0[:, :DH], x_n0[:, DH:]
    x1lo, x1hi = x_n1[:, :DH], x_n1[:, DH:]

    pad = jnp.concatenate(
        [jnp.ones((E, 1), jnp.float32), jnp.zeros((E, DEA - DE - 1), jnp.float32)],
        axis=1)
    ef0a = jnp.concatenate([edge_feature_0, pad], axis=1)
    ef1a = jnp.concatenate([edge_feature_1, pad], axis=1)

    sx, sea = _sc_segment_sums(src0, dst0, ef0a, x0lo, x0hi,
                               src1, dst1, ef1a, x1lo, x1hi)

    w10, w20, we0, cv0 = _combine(Wn0, Ws0, Wu0, bn0, bs0)
    w11, w21, we1, cv1 = _combine(Wn1, Ws1, Wu1, bn1, bs1)

    emb_n1 = _dense_out(sx[0:N], sx[N:2 * N], sea[0:N], x_n1,
                        w10, w20, we0, cv0, bu0.reshape(1, DOUT))
    emb_n0 = _dense_out(sx[2 * N:3 * N], sx[3 * N:], sea[N:], x_n0,
                        w11, w21, we1, cv1, bu1.reshape(1, DOUT))
    return (emb_n0, emb_n1)


# speedup vs baseline: 2.0464x; 1.0951x over previous
import jax
import jax.numpy as jnp
from jax import lax
from jax.experimental import pallas as pl
from jax.experimental.pallas import tpu as pltpu
from jax.experimental.pallas import tpu_sc as plsc

N = 10000
E = 160000
D = 256
DH = 128
DE = 16
DEA = 128
DOUT = 256

NTILES = 16
EPT = E // NTILES
B = 80
NBLK = EPT // B
RPT = 632


LAST = N - (NTILES - 1) * RPT


def _rowcopy(s, src_slot, dst_slot):
    @pl.when(s < NTILES - 1)
    def _():
        pltpu.sync_copy(src_slot(s * RPT, RPT), dst_slot(s * RPT, RPT))

    @pl.when(s == NTILES - 1)
    def _():
        base = (NTILES - 1) * RPT
        pltpu.sync_copy(src_slot(base, LAST), dst_slot(base, LAST))


def _sc_x_body(srcadj, dsts, xt, zx, out_x, acc_x,
               srci0, srci1, dsti0, dsti1, rows0, rows1,
               sem0, sem1, dsem0, dsem1):
    c = lax.axis_index("c")
    s = lax.axis_index("s")

    for h in range(2):
        p = 2 * c + h
        _rowcopy(s, lambda o, n: zx.at[pl.ds(0, n)],
                 lambda o, n: acc_x.at[pl.ds(o, n)])
        plsc.subcore_barrier()

        slots = ((srci0, dsti0, rows0, sem0, dsem0),
                 (srci1, dsti1, rows1, sem1, dsem1))

        def start(i, slot):
            srci, dsti, rows, sem, dsem = slots[slot]
            ebase = pl.multiple_of(c * E + s * EPT + i * B, 8)
            pltpu.async_copy(dsts.at[pl.ds(ebase, B)], dsti, dsem)
            base = pl.multiple_of(p * E + s * EPT + i * B, 8)
            pltpu.sync_copy(srcadj.at[pl.ds(base, B)], srci)
            pltpu.async_copy(xt.at[srci], rows, sem)

        def finish(i, slot):
            srci, dsti, rows, sem, dsem = slots[slot]
            ebase = pl.multiple_of(c * E + s * EPT + i * B, 8)
            pltpu.make_async_copy(dsts.at[pl.ds(ebase, B)], dsti, dsem).wait()
            pltpu.make_async_copy(xt.at[srci], rows, sem).wait()
            pltpu.sync_copy(rows, acc_x.at[dsti], add=True)

        start(0, 0)

        def blk(j, carry):
            i = 2 * j

            @pl.when(i + 1 < NBLK)
            def _():
                start(i + 1, 1)

            finish(i, 0)

            @pl.when(i + 2 < NBLK)
            def _():
                start(i + 2, 0)

            @pl.when(i + 1 < NBLK)
            def _():
                finish(i + 1, 1)

            return carry

        lax.fori_loop(0, (NBLK + 1) // 2, blk, 0)
        plsc.subcore_barrier()
        _rowcopy(s, lambda o, n: acc_x.at[pl.ds(o, n)],
                 lambda o, n: out_x.at[pl.ds(pl.multiple_of(p * N + o, 8), n)])
        plsc.subcore_barrier()


def _sc_e_body(dsts, efa, ze, out_e, acc_e,
               dsti0, dsti1, efb0, efb1, sem0, sem1):
    c = lax.axis_index("c")
    s = lax.axis_index("s")

    _rowcopy(s, lambda o, n: ze.at[pl.ds(0, n)],
             lambda o, n: acc_e.at[pl.ds(o, n)])
    plsc.subcore_barrier()

    slots = ((dsti0, efb0, sem0), (dsti1, efb1, sem1))

    def start(i, slot):
        dsti, efb, sem = slots[slot]
        ebase = pl.multiple_of(c * E + s * EPT + i * B, 8)
        pltpu.sync_copy(dsts.at[pl.ds(ebase, B)], dsti)
        pltpu.async_copy(efa.at[pl.ds(ebase, B)], efb, sem)

    def finish(i, slot):
        dsti, efb, sem = slots[slot]
        ebase = pl.multiple_of(c * E + s * EPT + i * B, 8)
        pltpu.make_async_copy(efa.at[pl.ds(ebase, B)], efb, sem).wait()
        pltpu.sync_copy(efb, acc_e.at[dsti], add=True)

    start(0, 0)

    def blk(j, carry):
        i = 2 * j

        @pl.when(i + 1 < NBLK)
        def _():
            start(i + 1, 1)

        finish(i, 0)

        @pl.when(i + 2 < NBLK)
        def _():
            start(i + 2, 0)

        @pl.when(i + 1 < NBLK)
        def _():
            finish(i + 1, 1)

        return carry

    lax.fori_loop(0, (NBLK + 1) // 2, blk, 0)
    plsc.subcore_barrier()
    _rowcopy(s, lambda o, n: acc_e.at[pl.ds(o, n)],
             lambda o, n: out_e.at[pl.ds(pl.multiple_of(c * N + o, 8), n)])


def _sc_segment_sums(src0, dst0, ef0a, x0lo, x0hi,
                     src1, dst1, ef1a, x1lo, x1hi):
    xt = jnp.concatenate([x0lo, x0hi, x1lo, x1hi], axis=0)
    srcadj = jnp.concatenate(
        [src0, src0 + N, src1 + 2 * N, src1 + 3 * N])
    dsts = jnp.concatenate([dst0, dst1])
    efa = jnp.concatenate([ef0a, ef1a], axis=0)

    zx = jnp.zeros((RPT, DH), jnp.float32)
    ze = jnp.zeros((RPT, DEA), jnp.float32)
    mesh = plsc.VectorSubcoreMesh(core_axis_name="c", subcore_axis_name="s")
    fx = pl.kernel(
        _sc_x_body,
        out_type=jax.ShapeDtypeStruct((4 * N, DH), jnp.float32),
        mesh=mesh,
        scratch_types=[
            pltpu.VMEM_SHARED((N, DH), jnp.float32),
            pltpu.VMEM((B,), jnp.int32),
            pltpu.VMEM((B,), jnp.int32),
            pltpu.VMEM((B,), jnp.int32),
            pltpu.VMEM((B,), jnp.int32),
            pltpu.VMEM((B, DH), jnp.float32),
            pltpu.VMEM((B, DH), jnp.float32),
            pltpu.SemaphoreType.DMA,
            pltpu.SemaphoreType.DMA,
            pltpu.SemaphoreType.DMA,
            pltpu.SemaphoreType.DMA,
        ],
    )
    fe = pl.kernel(
        _sc_e_body,
        out_type=jax.ShapeDtypeStruct((2 * N, DEA), jnp.float32),
        mesh=mesh,
        scratch_types=[
            pltpu.VMEM_SHARED((N, DEA), jnp.float32),
            pltpu.VMEM((B,), jnp.int32),
            pltpu.VMEM((B,), jnp.int32),
            pltpu.VMEM((B, DEA), jnp.float32),
            pltpu.VMEM((B, DEA), jnp.float32),
            pltpu.SemaphoreType.DMA,
            pltpu.SemaphoreType.DMA,
        ],
    )
    return fx(srcadj, dsts, xt, zx), fe(dsts, efa, ze)


def _combine_body(wn, ws, wu, bn, bs, w1, w2, we, cv):
    wu_n = wu[0:DOUT, :]
    wu_e = wu[DOUT:DOUT + DE, :]
    wu_s = wu[DOUT + DE:, :]
    w1[...] = jnp.dot(wn[...], wu_n, preferred_element_type=jnp.float32)
    w2[...] = jnp.dot(ws[...], wu_s, preferred_element_type=jnp.float32)
    we[...] = wu_e
    cv[...] = (jnp.dot(bn[...], wu_n, preferred_element_type=jnp.float32)
               + jnp.dot(bs[...], wu_s, preferred_element_type=jnp.float32))


def _combine(wn, ws, wu, bn, bs):
    return pl.pallas_call(
        _combine_body,
        out_shape=(
            jax.ShapeDtypeStruct((D, DOUT), jnp.float32),
            jax.ShapeDtypeStruct((D, DOUT), jnp.float32),
            jax.ShapeDtypeStruct((DE, DOUT), jnp.float32),
            jax.ShapeDtypeStruct((1, DOUT), jnp.float32),
        ),
    )(wn, ws, wu, bn.reshape(1, D), bs.reshape(1, D))


RB = 400


def _out_body(sx_lo, sx_hi, sea, xs, w1, w2, we, cv, bu, out):
    se = sea[:, 0:DE]
    deg = sea[:, DE:DE + 1]
    acc = jnp.dot(sx_lo[...], w1[0:DH, :], preferred_element_type=jnp.float32)
    acc += jnp.dot(sx_hi[...], w1[DH:, :], preferred_element_type=jnp.float32)
    acc += jnp.dot(xs[...] * deg, w2[...], preferred_element_type=jnp.float32)
    acc += jnp.dot(se, we[...], preferred_element_type=jnp.float32)
    acc += deg * cv[...]
    acc += bu[...]
    out[...] = acc


def _dense_out(sx_lo, sx_hi, sea, xs, w1, w2, we, cv, bu):
    grid = (N // RB,)
    row = lambda i: (i, 0)
    fix = lambda i: (0, 0)
    return pl.pallas_call(
        _out_body,
        grid=grid,
        in_specs=[
            pl.BlockSpec((RB, DH), row),
            pl.BlockSpec((RB, DH), row),
            pl.BlockSpec((RB, DEA), row),
            pl.BlockSpec((RB, D), row),
            pl.BlockSpec((D, DOUT), fix),
            pl.BlockSpec((D, DOUT), fix),
            pl.BlockSpec((DE, DOUT), fix),
            pl.BlockSpec((1, DOUT), fix),
            pl.BlockSpec((1, DOUT), fix),
        ],
        out_specs=pl.BlockSpec((RB, DOUT), row),
        out_shape=jax.ShapeDtypeStruct((N, DOUT), jnp.float32),
    )(sx_lo, sx_hi, sea, xs, w1, w2, we, cv, bu)


def kernel(x_n0, x_n1, edge_index_0, edge_feature_0,
           edge_index_1, edge_feature_1,
           Wn0, bn0, Ws0, bs0, Wu0, bu0,
           Wn1, bn1, Ws1, bs1, Wu1, bu1):
    src0 = edge_index_0[0].astype(jnp.int32)
    dst0 = edge_index_0[1].astype(jnp.int32)
    src1 = edge_index_1[0].astype(jnp.int32)
    dst1 = edge_index_1[1].astype(jnp.int32)

    x0lo, x0hi = x_n0[:, :DH], x_n0[:, DH:]
    x1lo, x1hi = x_n1[:, :DH], x_n1[:, DH:]

    pad = jnp.concatenate(
        [jnp.ones((E, 1), jnp.float32), jnp.zeros((E, DEA - DE - 1), jnp.float32)],
        axis=1)
    ef0a = jnp.concatenate([edge_feature_0, pad], axis=1)
    ef1a = jnp.concatenate([edge_feature_1, pad], axis=1)

    sx, sea = _sc_segment_sums(src0, dst0, ef0a, x0lo, x0hi,
                               src1, dst1, ef1a, x1lo, x1hi)

    w10, w20, we0, cv0 = _combine(Wn0, Ws0, Wu0, bn0, bs0)
    w11, w21, we1, cv1 = _combine(Wn1, Ws1, Wu1, bn1, bs1)

    emb_n1 = _dense_out(sx[0:N], sx[N:2 * N], sea[0:N], x_n1,
                        w10, w20, we0, cv0, bu0.reshape(1, DOUT))
    emb_n0 = _dense_out(sx[2 * N:3 * N], sx[3 * N:], sea[N:], x_n0,
                        w11, w21, we1, cv1, bu1.reshape(1, DOUT))
    return (emb_n0, emb_n1)


# speedup vs baseline: 2.1219x; 1.0369x over previous
import jax
import jax.numpy as jnp
from jax import lax
from jax.experimental import pallas as pl
from jax.experimental.pallas import tpu as pltpu
from jax.experimental.pallas import tpu_sc as plsc

N = 10000
E = 160000
D = 256
DH = 128
DE = 16
DEA = 128
DOUT = 256

NTILES = 16
EPT = E // NTILES
B = 80
NBLK = EPT // B
RPT = 632


LAST = N - (NTILES - 1) * RPT


def _rowcopy(s, src_slot, dst_slot):
    @pl.when(s < NTILES - 1)
    def _():
        pltpu.sync_copy(src_slot(s * RPT, RPT), dst_slot(s * RPT, RPT))

    @pl.when(s == NTILES - 1)
    def _():
        base = (NTILES - 1) * RPT
        pltpu.sync_copy(src_slot(base, LAST), dst_slot(base, LAST))


def _sc_x_body(srcadj, dsts, xt, zx, out_x, acc_x,
               srci0, srci1, dsti0, dsti1, rows0, rows1,
               sem0, sem1, dsem0, dsem1):
    c = lax.axis_index("c")
    s = lax.axis_index("s")

    for h in range(2):
        p = 2 * c + h
        _rowcopy(s, lambda o, n: zx.at[pl.ds(0, n)],
                 lambda o, n: acc_x.at[pl.ds(o, n)])
        plsc.subcore_barrier()

        slots = ((srci0, dsti0, rows0, sem0, dsem0),
                 (srci1, dsti1, rows1, sem1, dsem1))

        def start(i, slot):
            srci, dsti, rows, sem, dsem = slots[slot]
            ebase = pl.multiple_of(c * E + s * EPT + i * B, 8)
            pltpu.async_copy(dsts.at[pl.ds(ebase, B)], dsti, dsem)
            base = pl.multiple_of(p * E + s * EPT + i * B, 8)
            pltpu.sync_copy(srcadj.at[pl.ds(base, B)], srci)
            pltpu.async_copy(xt.at[srci], rows, sem)

        def finish(i, slot):
            srci, dsti, rows, sem, dsem = slots[slot]
            ebase = pl.multiple_of(c * E + s * EPT + i * B, 8)
            pltpu.make_async_copy(dsts.at[pl.ds(ebase, B)], dsti, dsem).wait()
            pltpu.make_async_copy(xt.at[srci], rows, sem).wait()
            pltpu.sync_copy(rows, acc_x.at[dsti], add=True)

        start(0, 0)

        def blk(j, carry):
            i = 2 * j

            @pl.when(i + 1 < NBLK)
            def _():
                start(i + 1, 1)

            finish(i, 0)

            @pl.when(i + 2 < NBLK)
            def _():
                start(i + 2, 0)

            @pl.when(i + 1 < NBLK)
            def _():
                finish(i + 1, 1)

            return carry

        lax.fori_loop(0, (NBLK + 1) // 2, blk, 0)
        plsc.subcore_barrier()
        _rowcopy(s, lambda o, n: acc_x.at[pl.ds(o, n)],
                 lambda o, n: out_x.at[pl.ds(pl.multiple_of(p * N + o, 8), n)])
        plsc.subcore_barrier()


def _sc_e_body(dsts, efa, ze, out_e, acc_e,
               dsti0, dsti1, efb0, efb1, sem0, sem1, dsem0, dsem1):
    c = lax.axis_index("c")
    s = lax.axis_index("s")

    _rowcopy(s, lambda o, n: ze.at[pl.ds(0, n)],
             lambda o, n: acc_e.at[pl.ds(o, n)])
    plsc.subcore_barrier()

    slots = ((dsti0, efb0, sem0, dsem0), (dsti1, efb1, sem1, dsem1))

    def start(i, slot):
        dsti, efb, sem, dsem = slots[slot]
        ebase = pl.multiple_of(c * E + s * EPT + i * B, 8)
        pltpu.async_copy(dsts.at[pl.ds(ebase, B)], dsti, dsem)
        pltpu.async_copy(efa.at[pl.ds(ebase, B)], efb, sem)

    def finish(i, slot):
        dsti, efb, sem, dsem = slots[slot]
        ebase = pl.multiple_of(c * E + s * EPT + i * B, 8)
        pltpu.make_async_copy(dsts.at[pl.ds(ebase, B)], dsti, dsem).wait()
        pltpu.make_async_copy(efa.at[pl.ds(ebase, B)], efb, sem).wait()
        pltpu.sync_copy(efb, acc_e.at[dsti], add=True)

    start(0, 0)

    def blk(j, carry):
        i = 2 * j

        @pl.when(i + 1 < NBLK)
        def _():
            start(i + 1, 1)

        finish(i, 0)

        @pl.when(i + 2 < NBLK)
        def _():
            start(i + 2, 0)

        @pl.when(i + 1 < NBLK)
        def _():
            finish(i + 1, 1)

        return carry

    lax.fori_loop(0, (NBLK + 1) // 2, blk, 0)
    plsc.subcore_barrier()
    _rowcopy(s, lambda o, n: acc_e.at[pl.ds(o, n)],
             lambda o, n: out_e.at[pl.ds(pl.multiple_of(c * N + o, 8), n)])


def _sc_segment_sums(src0, dst0, ef0a, x0lo, x0hi,
                     src1, dst1, ef1a, x1lo, x1hi):
    xt = jnp.concatenate([x0lo, x0hi, x1lo, x1hi], axis=0)
    srcadj = jnp.concatenate(
        [src0, src0 + N, src1 + 2 * N, src1 + 3 * N])
    dsts = jnp.concatenate([dst0, dst1])
    efa = jnp.concatenate([ef0a, ef1a], axis=0)

    zx = jnp.zeros((RPT, DH), jnp.float32)
    ze = jnp.zeros((RPT, DEA), jnp.float32)
    mesh = plsc.VectorSubcoreMesh(core_axis_name="c", subcore_axis_name="s")
    fx = pl.kernel(
        _sc_x_body,
        out_type=jax.ShapeDtypeStruct((4 * N, DH), jnp.float32),
        mesh=mesh,
        scratch_types=[
            pltpu.VMEM_SHARED((N, DH), jnp.float32),
            pltpu.VMEM((B,), jnp.int32),
            pltpu.VMEM((B,), jnp.int32),
            pltpu.VMEM((B,), jnp.int32),
            pltpu.VMEM((B,), jnp.int32),
            pltpu.VMEM((B, DH), jnp.float32),
            pltpu.VMEM((B, DH), jnp.float32),
            pltpu.SemaphoreType.DMA,
            pltpu.SemaphoreType.DMA,
            pltpu.SemaphoreType.DMA,
            pltpu.SemaphoreType.DMA,
        ],
    )
    fe = pl.kernel(
        _sc_e_body,
        out_type=jax.ShapeDtypeStruct((2 * N, DEA), jnp.float32),
        mesh=mesh,
        scratch_types=[
            pltpu.VMEM_SHARED((N, DEA), jnp.float32),
            pltpu.VMEM((B,), jnp.int32),
            pltpu.VMEM((B,), jnp.int32),
            pltpu.VMEM((B, DEA), jnp.float32),
            pltpu.VMEM((B, DEA), jnp.float32),
            pltpu.SemaphoreType.DMA,
            pltpu.SemaphoreType.DMA,
            pltpu.SemaphoreType.DMA,
            pltpu.SemaphoreType.DMA,
        ],
    )
    return fx(srcadj, dsts, xt, zx), fe(dsts, efa, ze)


def _combine_body(wn, ws, wu, bn, bs, w1, w2, we, cv):
    wu_n = wu[0:DOUT, :]
    wu_e = wu[DOUT:DOUT + DE, :]
    wu_s = wu[DOUT + DE:, :]
    w1[...] = jnp.dot(wn[...], wu_n, preferred_element_type=jnp.float32)
    w2[...] = jnp.dot(ws[...], wu_s, preferred_element_type=jnp.float32)
    we[...] = wu_e
    cv[...] = (jnp.dot(bn[...], wu_n, preferred_element_type=jnp.float32)
               + jnp.dot(bs[...], wu_s, preferred_element_type=jnp.float32))


def _combine(wn, ws, wu, bn, bs):
    return pl.pallas_call(
        _combine_body,
        out_shape=(
            jax.ShapeDtypeStruct((D, DOUT), jnp.float32),
            jax.ShapeDtypeStruct((D, DOUT), jnp.float32),
            jax.ShapeDtypeStruct((DE, DOUT), jnp.float32),
            jax.ShapeDtypeStruct((1, DOUT), jnp.float32),
        ),
    )(wn, ws, wu, bn.reshape(1, D), bs.reshape(1, D))


RB = 400


def _out_body(sx_lo, sx_hi, sea, xs, w1, w2, we, cv, bu, out):
    se = sea[:, 0:DE]
    deg = sea[:, DE:DE + 1]
    acc = jnp.dot(sx_lo[...], w1[0:DH, :], preferred_element_type=jnp.float32)
    acc += jnp.dot(sx_hi[...], w1[DH:, :], preferred_element_type=jnp.float32)
    acc += jnp.dot(xs[...] * deg, w2[...], preferred_element_type=jnp.float32)
    acc += jnp.dot(se, we[...], preferred_element_type=jnp.float32)
    acc += deg * cv[...]
    acc += bu[...]
    out[...] = acc


def _dense_out(sx_lo, sx_hi, sea, xs, w1, w2, we, cv, bu):
    grid = (N // RB,)
    row = lambda i: (i, 0)
    fix = lambda i: (0, 0)
    return pl.pallas_call(
        _out_body,
        grid=grid,
        in_specs=[
            pl.BlockSpec((RB, DH), row),
            pl.BlockSpec((RB, DH), row),
            pl.BlockSpec((RB, DEA), row),
            pl.BlockSpec((RB, D), row),
            pl.BlockSpec((D, DOUT), fix),
            pl.BlockSpec((D, DOUT), fix),
            pl.BlockSpec((DE, DOUT), fix),
            pl.BlockSpec((1, DOUT), fix),
            pl.BlockSpec((1, DOUT), fix),
        ],
        out_specs=pl.BlockSpec((RB, DOUT), row),
        out_shape=jax.ShapeDtypeStruct((N, DOUT), jnp.float32),
    )(sx_lo, sx_hi, sea, xs, w1, w2, we, cv, bu)


def kernel(x_n0, x_n1, edge_index_0, edge_feature_0,
           edge_index_1, edge_feature_1,
           Wn0, bn0, Ws0, bs0, Wu0, bu0,
           Wn1, bn1, Ws1, bs1, Wu1, bu1):
    src0 = edge_index_0[0].astype(jnp.int32)
    dst0 = edge_index_0[1].astype(jnp.int32)
    src1 = edge_index_1[0].astype(jnp.int32)
    dst1 = edge_index_1[1].astype(jnp.int32)

    x0lo, x0hi = x_n0[:, :DH], x_n0[:, DH:]
    x1lo, x1hi = x_n1[:, :DH], x_n1[:, DH:]

    pad = jnp.concatenate(
        [jnp.ones((E, 1), jnp.float32), jnp.zeros((E, DEA - DE - 1), jnp.float32)],
        axis=1)
    ef0a = jnp.concatenate([edge_feature_0, pad], axis=1)
    ef1a = jnp.concatenate([edge_feature_1, pad], axis=1)

    sx, sea = _sc_segment_sums(src0, dst0, ef0a, x0lo, x0hi,
                               src1, dst1, ef1a, x1lo, x1hi)

    w10, w20, we0, cv0 = _combine(Wn0, Ws0, Wu0, bn0, bs0)
    w11, w21, we1, cv1 = _combine(Wn1, Ws1, Wu1, bn1, bs1)

    emb_n1 = _dense_out(sx[0:N], sx[N:2 * N], sea[0:N], x_n1,
                        w10, w20, we0, cv0, bu0.reshape(1, DOUT))
    emb_n0 = _dense_out(sx[2 * N:3 * N], sx[3 * N:], sea[N:], x_n0,
                        w11, w21, we1, cv1, bu1.reshape(1, DOUT))
    return (emb_n0, emb_n1)


# speedup vs baseline: 2.2337x; 1.0527x over previous
import jax
import jax.numpy as jnp
from jax import lax
from jax.experimental import pallas as pl
from jax.experimental.pallas import tpu as pltpu
from jax.experimental.pallas import tpu_sc as plsc

N = 10000
E = 160000
D = 256
DH = 128
DE = 16
DEA = 128
DOUT = 256

NTILES = 16
EPT = E // NTILES
B = 80
NBLK = EPT // B
RPT = 632


LAST = N - (NTILES - 1) * RPT


def _rowcopy(s, src_slot, dst_slot):
    @pl.when(s < NTILES - 1)
    def _():
        pltpu.sync_copy(src_slot(s * RPT, RPT), dst_slot(s * RPT, RPT))

    @pl.when(s == NTILES - 1)
    def _():
        base = (NTILES - 1) * RPT
        pltpu.sync_copy(src_slot(base, LAST), dst_slot(base, LAST))


def _sc_x_body(srcadj, dsts, xt, zx, out_x, acc_x,
               srci0, srci1, srci2, dsti0, dsti1, dsti2,
               rows0, rows1, rows2,
               sem0, sem1, sem2, dsem0, dsem1, dsem2):
    c = lax.axis_index("c")
    s = lax.axis_index("s")

    for h in range(2):
        p = 2 * c + h
        _rowcopy(s, lambda o, n: zx.at[pl.ds(0, n)],
                 lambda o, n: acc_x.at[pl.ds(o, n)])
        plsc.subcore_barrier()

        slots = ((srci0, dsti0, rows0, sem0, dsem0),
                 (srci1, dsti1, rows1, sem1, dsem1),
                 (srci2, dsti2, rows2, sem2, dsem2))

        def start(i, slot):
            srci, dsti, rows, sem, dsem = slots[slot]
            ebase = pl.multiple_of(c * E + s * EPT + i * B, 8)
            pltpu.async_copy(dsts.at[pl.ds(ebase, B)], dsti, dsem)
            base = pl.multiple_of(p * E + s * EPT + i * B, 8)
            pltpu.sync_copy(srcadj.at[pl.ds(base, B)], srci)
            pltpu.async_copy(xt.at[srci], rows, sem)

        def finish(i, slot):
            srci, dsti, rows, sem, dsem = slots[slot]
            ebase = pl.multiple_of(c * E + s * EPT + i * B, 8)
            pltpu.make_async_copy(dsts.at[pl.ds(ebase, B)], dsti, dsem).wait()
            pltpu.make_async_copy(xt.at[srci], rows, sem).wait()
            pltpu.sync_copy(rows, acc_x.at[dsti], add=True)

        start(0, 0)
        start(1, 1)

        def blk(j, carry):
            i = 3 * j

            @pl.when(i + 2 < NBLK)
            def _():
                start(i + 2, 2)

            finish(i, 0)

            @pl.when(i + 3 < NBLK)
            def _():
                start(i + 3, 0)

            @pl.when(i + 1 < NBLK)
            def _():
                finish(i + 1, 1)

            @pl.when(i + 4 < NBLK)
            def _():
                start(i + 4, 1)

            @pl.when(i + 2 < NBLK)
            def _():
                finish(i + 2, 2)

            return carry

        lax.fori_loop(0, (NBLK + 2) // 3, blk, 0)
        plsc.subcore_barrier()
        _rowcopy(s, lambda o, n: acc_x.at[pl.ds(o, n)],
                 lambda o, n: out_x.at[pl.ds(pl.multiple_of(p * N + o, 8), n)])
        plsc.subcore_barrier()


def _sc_e_body(dsts, efa, ze, out_e, acc_e,
               dsti0, dsti1, efb0, efb1, sem0, sem1, dsem0, dsem1):
    c = lax.axis_index("c")
    s = lax.axis_index("s")

    _rowcopy(s, lambda o, n: ze.at[pl.ds(0, n)],
             lambda o, n: acc_e.at[pl.ds(o, n)])
    plsc.subcore_barrier()

    slots = ((dsti0, efb0, sem0, dsem0), (dsti1, efb1, sem1, dsem1))

    def start(i, slot):
        dsti, efb, sem, dsem = slots[slot]
        ebase = pl.multiple_of(c * E + s * EPT + i * B, 8)
        pltpu.async_copy(dsts.at[pl.ds(ebase, B)], dsti, dsem)
        pltpu.async_copy(efa.at[pl.ds(ebase, B)], efb, sem)

    def finish(i, slot):
        dsti, efb, sem, dsem = slots[slot]
        ebase = pl.multiple_of(c * E + s * EPT + i * B, 8)
        pltpu.make_async_copy(dsts.at[pl.ds(ebase, B)], dsti, dsem).wait()
        pltpu.make_async_copy(efa.at[pl.ds(ebase, B)], efb, sem).wait()
        pltpu.sync_copy(efb, acc_e.at[dsti], add=True)

    start(0, 0)

    def blk(j, carry):
        i = 2 * j

        @pl.when(i + 1 < NBLK)
        def _():
            start(i + 1, 1)

        finish(i, 0)

        @pl.when(i + 2 < NBLK)
        def _():
            start(i + 2, 0)

        @pl.when(i + 1 < NBLK)
        def _():
            finish(i + 1, 1)

        return carry

    lax.fori_loop(0, (NBLK + 1) // 2, blk, 0)
    plsc.subcore_barrier()
    _rowcopy(s, lambda o, n: acc_e.at[pl.ds(o, n)],
             lambda o, n: out_e.at[pl.ds(pl.multiple_of(c * N + o, 8), n)])


def _sc_segment_sums(src0, dst0, ef0a, x0lo, x0hi,
                     src1, dst1, ef1a, x1lo, x1hi):
    xt = jnp.concatenate([x0lo, x0hi, x1lo, x1hi], axis=0)
    srcadj = jnp.concatenate(
        [src0, src0 + N, src1 + 2 * N, src1 + 3 * N])
    dsts = jnp.concatenate([dst0, dst1])
    efa = jnp.concatenate([ef0a, ef1a], axis=0)

    zx = jnp.zeros((RPT, DH), jnp.float32)
    ze = jnp.zeros((RPT, DEA), jnp.float32)
    mesh = plsc.VectorSubcoreMesh(core_axis_name="c", subcore_axis_name="s")
    fx = pl.kernel(
        _sc_x_body,
        out_type=jax.ShapeDtypeStruct((4 * N, DH), jnp.float32),
        mesh=mesh,
        scratch_types=[
            pltpu.VMEM_SHARED((N, DH), jnp.float32),
            pltpu.VMEM((B,), jnp.int32),
            pltpu.VMEM((B,), jnp.int32),
            pltpu.VMEM((B,), jnp.int32),
            pltpu.VMEM((B,), jnp.int32),
            pltpu.VMEM((B,), jnp.int32),
            pltpu.VMEM((B,), jnp.int32),
            pltpu.VMEM((B, DH), jnp.float32),
            pltpu.VMEM((B, DH), jnp.float32),
            pltpu.VMEM((B, DH), jnp.float32),
            pltpu.SemaphoreType.DMA,
            pltpu.SemaphoreType.DMA,
            pltpu.SemaphoreType.DMA,
            pltpu.SemaphoreType.DMA,
            pltpu.SemaphoreType.DMA,
            pltpu.SemaphoreType.DMA,
        ],
    )
    fe = pl.kernel(
        _sc_e_body,
        out_type=jax.ShapeDtypeStruct((2 * N, DEA), jnp.float32),
        mesh=mesh,
        scratch_types=[
            pltpu.VMEM_SHARED((N, DEA), jnp.float32),
            pltpu.VMEM((B,), jnp.int32),
            pltpu.VMEM((B,), jnp.int32),
            pltpu.VMEM((B, DEA), jnp.float32),
            pltpu.VMEM((B, DEA), jnp.float32),
            pltpu.SemaphoreType.DMA,
            pltpu.SemaphoreType.DMA,
            pltpu.SemaphoreType.DMA,
            pltpu.SemaphoreType.DMA,
        ],
    )
    return fx(srcadj, dsts, xt, zx), fe(dsts, efa, ze)


def _combine_body(wn, ws, wu, bn, bs, w1, w2, we, cv):
    wu_n = wu[0:DOUT, :]
    wu_e = wu[DOUT:DOUT + DE, :]
    wu_s = wu[DOUT + DE:, :]
    w1[...] = jnp.dot(wn[...], wu_n, preferred_element_type=jnp.float32)
    w2[...] = jnp.dot(ws[...], wu_s, preferred_element_type=jnp.float32)
    we[...] = wu_e
    cv[...] = (jnp.dot(bn[...], wu_n, preferred_element_type=jnp.float32)
               + jnp.dot(bs[...], wu_s, preferred_element_type=jnp.float32))


def _combine(wn, ws, wu, bn, bs):
    return pl.pallas_call(
        _combine_body,
        out_shape=(
            jax.ShapeDtypeStruct((D, DOUT), jnp.float32),
            jax.ShapeDtypeStruct((D, DOUT), jnp.float32),
            jax.ShapeDtypeStruct((DE, DOUT), jnp.float32),
            jax.ShapeDtypeStruct((1, DOUT), jnp.float32),
        ),
    )(wn, ws, wu, bn.reshape(1, D), bs.reshape(1, D))


RB = 400


def _out_body(sx_lo, sx_hi, sea, xs, w1, w2, we, cv, bu, out):
    se = sea[:, 0:DE]
    deg = sea[:, DE:DE + 1]
    acc = jnp.dot(sx_lo[...], w1[0:DH, :], preferred_element_type=jnp.float32)
    acc += jnp.dot(sx_hi[...], w1[DH:, :], preferred_element_type=jnp.float32)
    acc += jnp.dot(xs[...] * deg, w2[...], preferred_element_type=jnp.float32)
    acc += jnp.dot(se, we[...], preferred_element_type=jnp.float32)
    acc += deg * cv[...]
    acc += bu[...]
    out[...] = acc


def _dense_out(sx_lo, sx_hi, sea, xs, w1, w2, we, cv, bu):
    grid = (N // RB,)
    row = lambda i: (i, 0)
    fix = lambda i: (0, 0)
    return pl.pallas_call(
        _out_body,
        grid=grid,
        in_specs=[
            pl.BlockSpec((RB, DH), row),
            pl.BlockSpec((RB, DH), row),
            pl.BlockSpec((RB, DEA), row),
            pl.BlockSpec((RB, D), row),
            pl.BlockSpec((D, DOUT), fix),
            pl.BlockSpec((D, DOUT), fix),
            pl.BlockSpec((DE, DOUT), fix),
            pl.BlockSpec((1, DOUT), fix),
            pl.BlockSpec((1, DOUT), fix),
        ],
        out_specs=pl.BlockSpec((RB, DOUT), row),
        out_shape=jax.ShapeDtypeStruct((N, DOUT), jnp.float32),
    )(sx_lo, sx_hi, sea, xs, w1, w2, we, cv, bu)


def kernel(x_n0, x_n1, edge_index_0, edge_feature_0,
           edge_index_1, edge_feature_1,
           Wn0, bn0, Ws0, bs0, Wu0, bu0,
           Wn1, bn1, Ws1, bs1, Wu1, bu1):
    src0 = edge_index_0[0].astype(jnp.int32)
    dst0 = edge_index_0[1].astype(jnp.int32)
    src1 = edge_index_1[0].astype(jnp.int32)
    dst1 = edge_index_1[1].astype(jnp.int32)

    x0lo, x0hi = x_n0[:, :DH], x_n0[:, DH:]
    x1lo, x1hi = x_n1[:, :DH], x_n1[:, DH:]

    pad = jnp.concatenate(
        [jnp.ones((E, 1), jnp.float32), jnp.zeros((E, DEA - DE - 1), jnp.float32)],
        axis=1)
    ef0a = jnp.concatenate([edge_feature_0, pad], axis=1)
    ef1a = jnp.concatenate([edge_feature_1, pad], axis=1)

    sx, sea = _sc_segment_sums(src0, dst0, ef0a, x0lo, x0hi,
                               src1, dst1, ef1a, x1lo, x1hi)

    w10, w20, we0, cv0 = _combine(Wn0, Ws0, Wu0, bn0, bs0)
    w11, w21, we1, cv1 = _combine(Wn1, Ws1, Wu1, bn1, bs1)

    emb_n1 = _dense_out(sx[0:N], sx[N:2 * N], sea[0:N], x_n1,
                        w10, w20, we0, cv0, bu0.reshape(1, DOUT))
    emb_n0 = _dense_out(sx[2 * N:3 * N], sx[3 * N:], sea[N:], x_n0,
                        w11, w21, we1, cv1, bu1.reshape(1, DOUT))
    return (emb_n0, emb_n1)
